# baseline (speedup 1.0000x reference)
import jax
import jax.numpy as jnp
from jax.experimental import pallas as pl
from jax.experimental.pallas import tpu as pltpu

_BB = 512


def _binarize_w_kernel(w1_ref, w2_ref, w1b_ref, w2b_ref):
    w1b_ref[...] = jnp.where(w1_ref[...] >= 0, 1.0, -1.0).astype(jnp.bfloat16)
    w2b_ref[...] = jnp.where(w2_ref[...] >= 0, 1.0, -1.0).astype(jnp.float8_e4m3fn)


_SPLIT = 2


def _mlp_kernel(x_ref, w1_ref, w2_ref, w3_ref,
                b1_ref, g1_ref, be1_ref, m1_ref, v1_ref,
                b2_ref, g2_ref, be2_ref, m2_ref, v2_ref,
                b3_ref, scale_ref, o_ref):
    dn = (((1,), (1,)), ((), ()))
    inv1 = g1_ref[...] * jax.lax.rsqrt(v1_ref[...] + 1e-5)
    c1 = be1_ref[...] - m1_ref[...] * inv1
    inv2 = g2_ref[...] * jax.lax.rsqrt(v2_ref[...] + 1e-5)
    c2 = be2_ref[...] - m2_ref[...] * inv2
    w3b = jnp.where(w3_ref[...] >= 0, 1.0, -1.0).astype(jnp.float8_e4m3fn)
    scale = scale_ref[0]

    sb = _BB // _SPLIT
    for s in range(_SPLIT):
        rs = slice(s * sb, (s + 1) * sb)
        xb = x_ref[rs, :].astype(jnp.bfloat16)
        z1 = jax.lax.dot_general(xb, w1_ref[...], dn,
                                 preferred_element_type=jnp.float32)
        bn1 = (z1 + b1_ref[...]) * inv1 + c1
        h1 = jnp.where(bn1 >= 0, 1.0, -1.0).astype(jnp.float8_e4m3fn)

        z2 = jax.lax.dot_general(h1, w2_ref[...], dn,
                                 preferred_element_type=jnp.float32)
        bn2 = (z2 + b2_ref[...]) * inv2 + c2
        h2 = jnp.where(bn2 >= 0, 1.0, -1.0).astype(jnp.float8_e4m3fn)

        z3 = jax.lax.dot_general(h2, w3b, dn,
                                 preferred_element_type=jnp.float32)
        o_ref[rs, :] = (z3 + b3_ref[...]) * scale


def kernel(x, w1, b1, g1, be1, m1, v1, w2, b2, g2, be2, m2, v2, w3, b3, scale):
    B = x.shape[0]
    H, D_IN = w1.shape
    D_OUT = w3.shape[0]
    x2 = x.reshape(B, D_IN)

    w1b, w2b = pl.pallas_call(
        _binarize_w_kernel,
        grid=(2,),
        in_specs=[
            pl.BlockSpec((H // 2, D_IN), lambda i: (i, 0)),
            pl.BlockSpec((H // 2, H), lambda i: (i, 0)),
        ],
        out_specs=[
            pl.BlockSpec((H // 2, D_IN), lambda i: (i, 0)),
            pl.BlockSpec((H // 2, H), lambda i: (i, 0)),
        ],
        out_shape=[
            jax.ShapeDtypeStruct((H, D_IN), jnp.bfloat16),
            jax.ShapeDtypeStruct((H, H), jnp.float8_e4m3fn),
        ],
        compiler_params=pltpu.CompilerParams(
            dimension_semantics=("parallel",),
        ),
        name="bnn_binarize_w",
    )(w1, w2)

    vrow = lambda a: a.reshape(1, -1)
    const = lambda i: (0, 0)
    out = pl.pallas_call(
        _mlp_kernel,
        grid=(B // _BB,),
        in_specs=[
            pl.BlockSpec((_BB, D_IN), lambda i: (i, 0)),
            pl.BlockSpec((H, D_IN), const),
            pl.BlockSpec((H, H), const),
            pl.BlockSpec((D_OUT, H), const),
            pl.BlockSpec((1, H), const),
            pl.BlockSpec((1, H), const),
            pl.BlockSpec((1, H), const),
            pl.BlockSpec((1, H), const),
            pl.BlockSpec((1, H), const),
            pl.BlockSpec((1, H), const),
            pl.BlockSpec((1, H), const),
            pl.BlockSpec((1, H), const),
            pl.BlockSpec((1, H), const),
            pl.BlockSpec((1, H), const),
            pl.BlockSpec((1, D_OUT), const),
            pl.BlockSpec(memory_space=pltpu.SMEM),
        ],
        out_specs=pl.BlockSpec((_BB, D_OUT), lambda i: (i, 0)),
        out_shape=jax.ShapeDtypeStruct((B, D_OUT), jnp.float32),
        compiler_params=pltpu.CompilerParams(
            dimension_semantics=("parallel",),
            vmem_limit_bytes=56 * 1024 * 1024,
        ),
        name="bnn_mlp_fused",
    )(x2, w1b, w2b, w3,
      vrow(b1), vrow(g1), vrow(be1), vrow(m1), vrow(v1),
      vrow(b2), vrow(g2), vrow(be2), vrow(m2), vrow(v2),
      vrow(b3), scale.reshape(1))
    return out


# speedup vs baseline: 1.1444x; 1.1444x over previous
import jax
import jax.numpy as jnp
from jax.experimental import pallas as pl
from jax.experimental.pallas import tpu as pltpu

_BB = 512


def _binarize_w_kernel(w1_ref, w2_ref, w1b_ref, w2b_ref):
    w1b_ref[...] = jnp.where(w1_ref[...] >= 0, 1.0, -1.0).astype(jnp.bfloat16)
    w2b_ref[...] = jnp.where(w2_ref[...] >= 0, 1.0, -1.0).astype(jnp.float8_e4m3fn)


_SPLIT = 2


def _mlp_kernel(x_ref, w1_ref, w2_ref, w3_ref,
                b1_ref, g1_ref, be1_ref, m1_ref, v1_ref,
                b2_ref, g2_ref, be2_ref, m2_ref, v2_ref,
                b3_ref, scale_ref, o_ref):
    dn = (((1,), (1,)), ((), ()))
    inv1 = g1_ref[...] * jax.lax.rsqrt(v1_ref[...] + 1e-5)
    c1 = be1_ref[...] - m1_ref[...] * inv1
    inv2 = g2_ref[...] * jax.lax.rsqrt(v2_ref[...] + 1e-5)
    c2 = be2_ref[...] - m2_ref[...] * inv2
    w3b = jnp.where(w3_ref[...] >= 0, 1.0, -1.0).astype(jnp.float8_e4m3fn)
    scale = scale_ref[0]

    sb = _BB // _SPLIT
    for s in range(_SPLIT):
        rs = slice(s * sb, (s + 1) * sb)
        xb = x_ref[:, rs]
        z1 = jax.lax.dot_general(xb, w1_ref[...],
                                 (((0,), (1,)), ((), ())),
                                 preferred_element_type=jnp.float32)
        bn1 = (z1 + b1_ref[...]) * inv1 + c1
        h1 = jnp.where(bn1 >= 0, 1.0, -1.0).astype(jnp.float8_e4m3fn)

        z2 = jax.lax.dot_general(h1, w2_ref[...], dn,
                                 preferred_element_type=jnp.float32)
        bn2 = (z2 + b2_ref[...]) * inv2 + c2
        h2 = jnp.where(bn2 >= 0, 1.0, -1.0).astype(jnp.float8_e4m3fn)

        z3 = jax.lax.dot_general(h2, w3b, dn,
                                 preferred_element_type=jnp.float32)
        o_ref[rs, :] = (z3 + b3_ref[...]) * scale


def kernel(x, w1, b1, g1, be1, m1, v1, w2, b2, g2, be2, m2, v2, w3, b3, scale):
    B = x.shape[0]
    H, D_IN = w1.shape
    D_OUT = w3.shape[0]
    x2t = x.reshape(B, D_IN).T.astype(jnp.bfloat16)

    w1b, w2b = pl.pallas_call(
        _binarize_w_kernel,
        grid=(2,),
        in_specs=[
            pl.BlockSpec((H // 2, D_IN), lambda i: (i, 0)),
            pl.BlockSpec((H // 2, H), lambda i: (i, 0)),
        ],
        out_specs=[
            pl.BlockSpec((H // 2, D_IN), lambda i: (i, 0)),
            pl.BlockSpec((H // 2, H), lambda i: (i, 0)),
        ],
        out_shape=[
            jax.ShapeDtypeStruct((H, D_IN), jnp.bfloat16),
            jax.ShapeDtypeStruct((H, H), jnp.float8_e4m3fn),
        ],
        compiler_params=pltpu.CompilerParams(
            dimension_semantics=("parallel",),
        ),
        name="bnn_binarize_w",
    )(w1, w2)

    vrow = lambda a: a.reshape(1, -1)
    const = lambda i: (0, 0)
    out = pl.pallas_call(
        _mlp_kernel,
        grid=(B // _BB,),
        in_specs=[
            pl.BlockSpec((D_IN, _BB), lambda i: (0, i)),
            pl.BlockSpec((H, D_IN), const),
            pl.BlockSpec((H, H), const),
            pl.BlockSpec((D_OUT, H), const),
            pl.BlockSpec((1, H), const),
            pl.BlockSpec((1, H), const),
            pl.BlockSpec((1, H), const),
            pl.BlockSpec((1, H), const),
            pl.BlockSpec((1, H), const),
            pl.BlockSpec((1, H), const),
            pl.BlockSpec((1, H), const),
            pl.BlockSpec((1, H), const),
            pl.BlockSpec((1, H), const),
            pl.BlockSpec((1, H), const),
            pl.BlockSpec((1, D_OUT), const),
            pl.BlockSpec(memory_space=pltpu.SMEM),
        ],
        out_specs=pl.BlockSpec((_BB, D_OUT), lambda i: (i, 0)),
        out_shape=jax.ShapeDtypeStruct((B, D_OUT), jnp.float32),
        compiler_params=pltpu.CompilerParams(
            dimension_semantics=("parallel",),
            vmem_limit_bytes=56 * 1024 * 1024,
        ),
        name="bnn_mlp_fused",
    )(x2t, w1b, w2b, w3,
      vrow(b1), vrow(g1), vrow(be1), vrow(m1), vrow(v1),
      vrow(b2), vrow(g2), vrow(be2), vrow(m2), vrow(v2),
      vrow(b3), scale.reshape(1))
    return out
